# baseline (speedup 1.0000x reference)
import dataclasses

import jax
import jax.numpy as jnp
from jax import lax
from jax.experimental import pallas as pl
from jax.experimental.pallas import tpu as pltpu
from jax.experimental.pallas import tpu_sc as plsc

CAP = 50000
HID = 768
K = 32
NC = 2
NS = 16
L = 16
RPT = 1568
CH = 48
IDXBUF = RPT + CH
EUN = 16

INF = float("inf")
IMAX = 0x7FFFFFFF


def _iota():
    return lax.iota(jnp.int32, L)


def _sort16(k, v):
    return plsc.sort_key_val(k, v)


def _rev(x):
    return jnp.flip(x, 0)


def _cmpx(ak, av, bk, bv):
    m = ak <= bk
    lk = jnp.where(m, ak, bk)
    lv = jnp.where(m, av, bv)
    hk = jnp.where(m, bk, ak)
    hv = jnp.where(m, bv, av)
    return lk, lv, hk, hv


def _merge_batch(b0k, b0v, b1k, b1v, nk, nv):
    nk, nv = _sort16(nk, nv)
    l1k, l1v, _, _ = _cmpx(b1k, b1v, _rev(nk), _rev(nv))
    l1k, l1v = _sort16(l1k, l1v)
    l2k, l2v, h2k, h2v = _cmpx(b0k, b0v, _rev(l1k), _rev(l1v))
    l2k, l2v = _sort16(l2k, l2v)
    h2k, h2v = _sort16(h2k, h2v)
    return l2k, l2v, h2k, h2v


def _merge32(a0k, a0v, a1k, a1v, p0k, p0v, p1k, p1v):
    l0k, l0v, _, _ = _cmpx(a0k, a0v, _rev(p1k), _rev(p1v))
    l1k, l1v, _, _ = _cmpx(a1k, a1v, _rev(p0k), _rev(p0v))
    l0k, l0v = _sort16(l0k, l0v)
    l1k, l1v = _sort16(l1k, l1v)
    m0k, m0v, m1k, m1v = _cmpx(l0k, l0v, _rev(l1k), _rev(l1v))
    m0k, m0v = _sort16(m0k, m0v)
    m1k, m1v = _sort16(m1k, m1v)
    return m0k, m0v, m1k, m1v


def _row_d2_contig(buf, qv):
    acc = jnp.zeros((L,), jnp.float32)
    for c in range(HID // L):
        d = buf[pl.ds(c * L, L)] - qv[pl.ds(c * L, L)]
        acc = acc + d * d
    return jnp.sum(acc)


def _phase1_kernel(q_hbm, bank_hbm, cent_hbm, cid_hbm, outk_hbm, outr_hbm,
                   qv, cb0, cb1, cstage, clocal, kbuf, cidv, idxv, rbuf,
                   mk, mi, pk, pi, iw, rbig, csh, mshk, mshi):
    c = lax.axis_index("c")
    s = lax.axis_index("s")
    wid = s * NC + c
    iota = _iota()

    pltpu.sync_copy(q_hbm, qv)

    pltpu.sync_copy(cent_hbm.at[2 * s], cb0)
    pltpu.sync_copy(cent_hbm.at[2 * s + 1], cb1)
    d2_0 = _row_d2_contig(cb0, qv)
    d2_1 = _row_d2_contig(cb1, qv)
    vrow = jnp.where(iota == 0, d2_0, jnp.where(iota == 1, d2_1, INF))
    cstage[...] = vrow
    pltpu.sync_copy(cstage, csh.at[pl.ds(pl.multiple_of(s * L, 8), L)])
    plsc.subcore_barrier()
    pltpu.sync_copy(csh, clocal)
    plsc.subcore_barrier()

    for s2 in range(NS):
        row = clocal[pl.ds(s2 * L, L)]
        key = lax.bitcast_convert_type(row, jnp.int32)
        key = jnp.bitwise_and(key, jnp.int32(~0x1F))
        key = jnp.bitwise_or(key, 2 * s2 + iota)
        key = jnp.where(iota < 2, key, IMAX)
        kbuf[pl.ds(s2 * L, L)] = key

    tsel = []
    for _ in range(3):
        km = kbuf[pl.ds(0, L)]
        for s2 in range(1, NS):
            km = jnp.minimum(km, kbuf[pl.ds(s2 * L, L)])
        kmin = jnp.min(km)
        tsel.append(jnp.bitwise_and(kmin, jnp.int32(0x1F)))
        for s2 in range(NS):
            row = kbuf[pl.ds(s2 * L, L)]
            kbuf[pl.ds(s2 * L, L)] = jnp.where(row == kmin, IMAX, row)
    t0, t1, t2 = tsel

    lo = wid * RPT
    base = lax.min(lo, CAP - RPT)
    pltpu.sync_copy(cid_hbm.at[pl.ds(pl.multiple_of(base, 8), RPT)], cidv)
    zeros16 = jnp.zeros((L,), jnp.int32)
    for j in range(IDXBUF // L):
        idxv[pl.ds(j * L, L)] = zeros16

    def comp_body(j, cnt):
        cvec = cidv[pl.ds(j * L, L)]
        gvec = base + j * L + iota
        m = (cvec == t0) | (cvec == t1) | (cvec == t2)
        m = m & (gvec >= lo)
        plsc.store_compressed(idxv.at[pl.ds(cnt, L)], gvec, mask=m)
        return cnt + jnp.sum(m.astype(jnp.int32))

    cnt = lax.fori_loop(0, RPT // L, comp_body, jnp.int32(0))

    infv = jnp.full((L,), INF, jnp.float32)
    zi = jnp.zeros((L,), jnp.int32)
    zf = jnp.zeros((L,), jnp.float32)
    nch = (cnt + (CH - 1)) // CH
    rows0 = iota
    rows1 = iota + L
    rows2 = iota + 2 * L

    def chunk_body(g, carry):
        b0k, b0v, b1k, b1v = carry
        off = pl.multiple_of(g * CH, 8)
        pltpu.sync_copy(bank_hbm.at[idxv.at[pl.ds(off, CH)]], rbuf)

        def eb(j, ec):
            a0, a1, a2, colw = ec
            for _ in range(EUN):
                qe = plsc.load_gather(qv, [colw])
                x0 = plsc.load_gather(rbuf, [rows0, colw])
                x1 = plsc.load_gather(rbuf, [rows1, colw])
                x2 = plsc.load_gather(rbuf, [rows2, colw])
                d0 = x0 - qe
                d1 = x1 - qe
                d2 = x2 - qe
                a0 = a0 + d0 * d0
                a1 = a1 + d1 * d1
                a2 = a2 + d2 * d2
                coln = colw + 1
                colw = jnp.where(coln == HID, 0, coln)
            return (a0, a1, a2, colw)

        a0, a1, a2, _ = lax.fori_loop(0, HID // EUN, eb, (zf, zf, zf, iota))
        for grp, acc in enumerate((a0, a1, a2)):
            ivec = idxv[pl.ds(pl.multiple_of(off + grp * L, 8), L)]
            pos = off + grp * L + iota
            dbatch = jnp.where(pos < cnt, acc, INF)
            b0k, b0v, b1k, b1v = _merge_batch(b0k, b0v, b1k, b1v, dbatch, ivec)
        return (b0k, b0v, b1k, b1v)

    b0k, b0v, b1k, b1v = lax.fori_loop(
        0, nch, chunk_body, (infv, zi, infv, zi))

    for r in (1, 2, 4, 8):
        mk[pl.ds(0, L)] = b0k
        mk[pl.ds(L, L)] = b1k
        mi[pl.ds(0, L)] = b0v
        mi[pl.ds(L, L)] = b1v
        pltpu.sync_copy(mk, mshk.at[pl.ds(pl.multiple_of(s * 2 * L, 8), 2 * L)])
        pltpu.sync_copy(mi, mshi.at[pl.ds(pl.multiple_of(s * 2 * L, 8), 2 * L)])
        plsc.subcore_barrier()

        def merged(operand, r=r):
            b0k, b0v, b1k, b1v = operand
            poff = pl.multiple_of((s + r) * 2 * L, 8)
            pltpu.sync_copy(mshk.at[pl.ds(poff, 2 * L)], pk)
            pltpu.sync_copy(mshi.at[pl.ds(poff, 2 * L)], pi)
            return _merge32(b0k, b0v, b1k, b1v,
                            pk[pl.ds(0, L)], pi[pl.ds(0, L)],
                            pk[pl.ds(L, L)], pi[pl.ds(L, L)])

        b0k, b0v, b1k, b1v = lax.cond(
            s % (2 * r) == 0, merged, lambda op: op, (b0k, b0v, b1k, b1v))
        plsc.subcore_barrier()

    @pl.when(s == 0)
    def _():
        mk[pl.ds(0, L)] = b0k
        mk[pl.ds(L, L)] = b1k
        iw[pl.ds(0, L)] = b0v
        iw[pl.ds(L, L)] = b1v
        off = pl.multiple_of(c * 2 * L, 8)
        pltpu.sync_copy(mk, outk_hbm.at[pl.ds(off, 2 * L)])
        pltpu.sync_copy(bank_hbm.at[iw], rbig)
        pltpu.sync_copy(rbig, outr_hbm.at[pl.ds(off, 2 * L)])


def _finish_kernel(kc_ref, kr_ref, rows_ref, out_ref):
    kc = kc_ref[...]
    kr = kr_ref[...]
    ic = lax.broadcasted_iota(jnp.int32, (2 * K, 2 * K), 0)
    ir = lax.broadcasted_iota(jnp.int32, (2 * K, 2 * K), 1)
    beats = (kr < kc) | ((kr == kc) & (ir < ic))
    rank = jnp.sum(beats.astype(jnp.int32), axis=1, keepdims=True)
    sel = lax.broadcasted_iota(jnp.int32, (2 * K, K), 1)
    onehot = (rank == sel).astype(jnp.float32)
    out_ref[...] = lax.dot_general(
        onehot, rows_ref[...], (((0,), (0,)), ((), ())),
        preferred_element_type=jnp.float32)


def _sc_params():
    cp = pltpu.CompilerParams()
    if "needs_layout_passes" in pltpu.CompilerParams.__dataclass_fields__:
        cp = dataclasses.replace(cp, needs_layout_passes=False)
    return cp


def kernel(query, memory_bank, centroids, cluster_ids, k):
    del k
    mesh = plsc.VectorSubcoreMesh(core_axis_name="c", subcore_axis_name="s")

    phase1 = pl.kernel(
        _phase1_kernel,
        out_type=(
            jax.ShapeDtypeStruct((NC * 2 * L,), jnp.float32),
            jax.ShapeDtypeStruct((NC * 2 * L, HID), jnp.float32),
        ),
        mesh=mesh,
        compiler_params=_sc_params(),
        scratch_types=[
            pltpu.VMEM((HID,), jnp.float32),
            pltpu.VMEM((HID,), jnp.float32),
            pltpu.VMEM((HID,), jnp.float32),
            pltpu.VMEM((L,), jnp.float32),
            pltpu.VMEM((NS * L,), jnp.float32),
            pltpu.VMEM((NS * L,), jnp.int32),
            pltpu.VMEM((RPT,), jnp.int32),
            pltpu.VMEM((IDXBUF,), jnp.int32),
            pltpu.VMEM((CH, HID), jnp.float32),
            pltpu.VMEM((2 * L,), jnp.float32),
            pltpu.VMEM((2 * L,), jnp.int32),
            pltpu.VMEM((2 * L,), jnp.float32),
            pltpu.VMEM((2 * L,), jnp.int32),
            pltpu.VMEM((2 * L,), jnp.int32),
            pltpu.VMEM((2 * L, HID), jnp.float32),
            pltpu.VMEM_SHARED((NS * L,), jnp.float32),
            pltpu.VMEM_SHARED((NS * 2 * L,), jnp.float32),
            pltpu.VMEM_SHARED((NS * 2 * L,), jnp.int32),
        ],
    )
    keys2, rows64 = phase1(query, memory_bank, centroids, cluster_ids)

    finish = pl.pallas_call(
        _finish_kernel,
        out_shape=jax.ShapeDtypeStruct((K, HID), jnp.float32),
    )
    return finish(keys2.reshape(2 * K, 1), keys2.reshape(1, 2 * K), rows64)


# speedup vs baseline: 1.0604x; 1.0604x over previous
import dataclasses

import jax
import jax.numpy as jnp
from jax import lax
from jax.experimental import pallas as pl
from jax.experimental.pallas import tpu as pltpu
from jax.experimental.pallas import tpu_sc as plsc

CAP = 50000
HID = 768
K = 32
NC = 2
NS = 16
L = 16
RPT = 1568
CH = 48
IDXBUF = RPT + CH
EUN = 16

INF = float("inf")
IMAX = 0x7FFFFFFF


def _iota():
    return lax.iota(jnp.int32, L)


def _sort16(k, v):
    return plsc.sort_key_val(k, v)


def _rev(x):
    return jnp.flip(x, 0)


def _cmpx(ak, av, bk, bv):
    m = ak <= bk
    lk = jnp.where(m, ak, bk)
    lv = jnp.where(m, av, bv)
    hk = jnp.where(m, bk, ak)
    hv = jnp.where(m, bv, av)
    return lk, lv, hk, hv


def _merge_batch(b0k, b0v, b1k, b1v, nk, nv):
    nk, nv = _sort16(nk, nv)
    l1k, l1v, _, _ = _cmpx(b1k, b1v, _rev(nk), _rev(nv))
    l1k, l1v = _sort16(l1k, l1v)
    l2k, l2v, h2k, h2v = _cmpx(b0k, b0v, _rev(l1k), _rev(l1v))
    l2k, l2v = _sort16(l2k, l2v)
    h2k, h2v = _sort16(h2k, h2v)
    return l2k, l2v, h2k, h2v


def _merge32(a0k, a0v, a1k, a1v, p0k, p0v, p1k, p1v):
    l0k, l0v, _, _ = _cmpx(a0k, a0v, _rev(p1k), _rev(p1v))
    l1k, l1v, _, _ = _cmpx(a1k, a1v, _rev(p0k), _rev(p0v))
    l0k, l0v = _sort16(l0k, l0v)
    l1k, l1v = _sort16(l1k, l1v)
    m0k, m0v, m1k, m1v = _cmpx(l0k, l0v, _rev(l1k), _rev(l1v))
    m0k, m0v = _sort16(m0k, m0v)
    m1k, m1v = _sort16(m1k, m1v)
    return m0k, m0v, m1k, m1v


def _row_d2_contig(buf, qv):
    acc = jnp.zeros((L,), jnp.float32)
    for c in range(HID // L):
        d = buf[pl.ds(c * L, L)] - qv[pl.ds(c * L, L)]
        acc = acc + d * d
    return jnp.sum(acc)


def _phase1_kernel(q_hbm, bank_hbm, cent_hbm, cid_hbm, outk_hbm, outr_hbm,
                   qv, cb0, cb1, cstage, clocal, kbuf, cidv, idxv,
                   rbufa, rbufb, sema, semb, semc,
                   mk, mi, pk, pi, iw, rbig, csh, mshk, mshi):
    c = lax.axis_index("c")
    s = lax.axis_index("s")
    wid = s * NC + c
    iota = _iota()

    lo = wid * RPT
    base = lax.min(lo, CAP - RPT)
    pltpu.async_copy(
        cid_hbm.at[pl.ds(pl.multiple_of(base, 8), RPT)], cidv, semc)
    pltpu.sync_copy(q_hbm, qv)

    pltpu.sync_copy(cent_hbm.at[2 * s], cb0)
    pltpu.sync_copy(cent_hbm.at[2 * s + 1], cb1)
    d2_0 = _row_d2_contig(cb0, qv)
    d2_1 = _row_d2_contig(cb1, qv)
    vrow = jnp.where(iota == 0, d2_0, jnp.where(iota == 1, d2_1, INF))
    cstage[...] = vrow
    pltpu.sync_copy(cstage, csh.at[pl.ds(pl.multiple_of(s * L, 8), L)])
    plsc.subcore_barrier()
    pltpu.sync_copy(csh, clocal)
    plsc.subcore_barrier()

    for s2 in range(NS):
        row = clocal[pl.ds(s2 * L, L)]
        key = lax.bitcast_convert_type(row, jnp.int32)
        key = jnp.bitwise_and(key, jnp.int32(~0x1F))
        key = jnp.bitwise_or(key, 2 * s2 + iota)
        key = jnp.where(iota < 2, key, IMAX)
        kbuf[pl.ds(s2 * L, L)] = key

    tsel = []
    for _ in range(3):
        km = kbuf[pl.ds(0, L)]
        for s2 in range(1, NS):
            km = jnp.minimum(km, kbuf[pl.ds(s2 * L, L)])
        kmin = jnp.min(km)
        tsel.append(jnp.bitwise_and(kmin, jnp.int32(0x1F)))
        for s2 in range(NS):
            row = kbuf[pl.ds(s2 * L, L)]
            kbuf[pl.ds(s2 * L, L)] = jnp.where(row == kmin, IMAX, row)
    t0, t1, t2 = tsel

    pltpu.make_async_copy(
        cid_hbm.at[pl.ds(pl.multiple_of(base, 8), RPT)], cidv, semc).wait()
    zeros16 = jnp.zeros((L,), jnp.int32)
    for j in range(IDXBUF // L):
        idxv[pl.ds(j * L, L)] = zeros16

    def comp_body(j, cnt):
        cvec = cidv[pl.ds(j * L, L)]
        gvec = base + j * L + iota
        m = (cvec == t0) | (cvec == t1) | (cvec == t2)
        m = m & (gvec >= lo)
        plsc.store_compressed(idxv.at[pl.ds(cnt, L)], gvec, mask=m)
        return cnt + jnp.sum(m.astype(jnp.int32))

    cnt = lax.fori_loop(0, RPT // L, comp_body, jnp.int32(0))

    infv = jnp.full((L,), INF, jnp.float32)
    zi = jnp.zeros((L,), jnp.int32)
    zf = jnp.zeros((L,), jnp.float32)
    nch = (cnt + (CH - 1)) // CH
    rows0 = iota
    rows1 = iota + L
    rows2 = iota + 2 * L

    def issue(g, rbuf, sem):
        off = pl.multiple_of(g * CH, 8)
        pltpu.async_copy(bank_hbm.at[idxv.at[pl.ds(off, CH)]], rbuf, sem)

    def wait(rbuf, sem):
        pltpu.make_async_copy(
            bank_hbm.at[idxv.at[pl.ds(0, CH)]], rbuf, sem).wait()

    def compute_merge(g, rbuf, carry):
        b0k, b0v, b1k, b1v = carry
        off = pl.multiple_of(g * CH, 8)

        def eb(j, ec):
            acc, cols = ec
            acc = list(acc)
            cols = list(cols)
            for u in range(EUN):
                p = u % 2
                colw = cols[p]
                qe = plsc.load_gather(qv, [colw])
                x0 = plsc.load_gather(rbuf, [rows0, colw])
                x1 = plsc.load_gather(rbuf, [rows1, colw])
                x2 = plsc.load_gather(rbuf, [rows2, colw])
                d0 = x0 - qe
                d1 = x1 - qe
                d2 = x2 - qe
                acc[p * 3 + 0] = acc[p * 3 + 0] + d0 * d0
                acc[p * 3 + 1] = acc[p * 3 + 1] + d1 * d1
                acc[p * 3 + 2] = acc[p * 3 + 2] + d2 * d2
                coln = colw + 2
                cols[p] = jnp.where(coln >= HID, coln - HID, coln)
            return (tuple(acc), tuple(cols))

        acc, _ = lax.fori_loop(0, HID // EUN, eb,
                               ((zf,) * 6, (iota, iota + 1)))
        for grp in range(CH // L):
            a = acc[grp] + acc[3 + grp]
            ivec = idxv[pl.ds(pl.multiple_of(off + grp * L, 8), L)]
            pos = off + grp * L + iota
            dbatch = jnp.where(pos < cnt, a, INF)
            b0k, b0v, b1k, b1v = _merge_batch(b0k, b0v, b1k, b1v, dbatch, ivec)
        return (b0k, b0v, b1k, b1v)

    @pl.when(nch > 0)
    def _():
        issue(0, rbufa, sema)

    def pair_body(t, carry):
        g0 = 2 * t
        wait(rbufa, sema)

        @pl.when(g0 + 1 < nch)
        def _():
            issue(g0 + 1, rbufb, semb)

        carry = compute_merge(g0, rbufa, carry)

        def second(carry):
            wait(rbufb, semb)

            @pl.when(g0 + 2 < nch)
            def _():
                issue(g0 + 2, rbufa, sema)

            return compute_merge(g0 + 1, rbufb, carry)

        return lax.cond(g0 + 1 < nch, second, lambda op: op, carry)

    b0k, b0v, b1k, b1v = lax.fori_loop(
        0, (nch + 1) // 2, pair_body, (infv, zi, infv, zi))

    for r in (1, 2, 4, 8):
        mk[pl.ds(0, L)] = b0k
        mk[pl.ds(L, L)] = b1k
        mi[pl.ds(0, L)] = b0v
        mi[pl.ds(L, L)] = b1v
        pltpu.sync_copy(mk, mshk.at[pl.ds(pl.multiple_of(s * 2 * L, 8), 2 * L)])
        pltpu.sync_copy(mi, mshi.at[pl.ds(pl.multiple_of(s * 2 * L, 8), 2 * L)])
        plsc.subcore_barrier()

        def merged(operand, r=r):
            b0k, b0v, b1k, b1v = operand
            poff = pl.multiple_of((s + r) * 2 * L, 8)
            pltpu.sync_copy(mshk.at[pl.ds(poff, 2 * L)], pk)
            pltpu.sync_copy(mshi.at[pl.ds(poff, 2 * L)], pi)
            return _merge32(b0k, b0v, b1k, b1v,
                            pk[pl.ds(0, L)], pi[pl.ds(0, L)],
                            pk[pl.ds(L, L)], pi[pl.ds(L, L)])

        b0k, b0v, b1k, b1v = lax.cond(
            s % (2 * r) == 0, merged, lambda op: op, (b0k, b0v, b1k, b1v))
        plsc.subcore_barrier()

    @pl.when(s == 0)
    def _():
        mk[pl.ds(0, L)] = b0k
        mk[pl.ds(L, L)] = b1k
        iw[pl.ds(0, L)] = b0v
        iw[pl.ds(L, L)] = b1v
        off = pl.multiple_of(c * 2 * L, 8)
        pltpu.sync_copy(mk, outk_hbm.at[pl.ds(off, 2 * L)])
        pltpu.sync_copy(bank_hbm.at[iw], rbig)
        pltpu.sync_copy(rbig, outr_hbm.at[pl.ds(off, 2 * L)])


def _finish_kernel(kc_ref, kr_ref, rows_ref, out_ref):
    kc = kc_ref[...]
    kr = kr_ref[...]
    ic = lax.broadcasted_iota(jnp.int32, (2 * K, 2 * K), 0)
    ir = lax.broadcasted_iota(jnp.int32, (2 * K, 2 * K), 1)
    beats = (kr < kc) | ((kr == kc) & (ir < ic))
    rank = jnp.sum(beats.astype(jnp.int32), axis=1, keepdims=True)
    sel = lax.broadcasted_iota(jnp.int32, (2 * K, K), 1)
    onehot = (rank == sel).astype(jnp.float32)
    out_ref[...] = lax.dot_general(
        onehot, rows_ref[...], (((0,), (0,)), ((), ())),
        precision=lax.Precision.HIGHEST,
        preferred_element_type=jnp.float32)


def _sc_params():
    cp = pltpu.CompilerParams()
    if "needs_layout_passes" in pltpu.CompilerParams.__dataclass_fields__:
        cp = dataclasses.replace(cp, needs_layout_passes=False)
    return cp


def kernel(query, memory_bank, centroids, cluster_ids, k):
    del k
    mesh = plsc.VectorSubcoreMesh(core_axis_name="c", subcore_axis_name="s")

    phase1 = pl.kernel(
        _phase1_kernel,
        out_type=(
            jax.ShapeDtypeStruct((NC * 2 * L,), jnp.float32),
            jax.ShapeDtypeStruct((NC * 2 * L, HID), jnp.float32),
        ),
        mesh=mesh,
        compiler_params=_sc_params(),
        scratch_types=[
            pltpu.VMEM((HID,), jnp.float32),
            pltpu.VMEM((HID,), jnp.float32),
            pltpu.VMEM((HID,), jnp.float32),
            pltpu.VMEM((L,), jnp.float32),
            pltpu.VMEM((NS * L,), jnp.float32),
            pltpu.VMEM((NS * L,), jnp.int32),
            pltpu.VMEM((RPT,), jnp.int32),
            pltpu.VMEM((IDXBUF,), jnp.int32),
            pltpu.VMEM((CH, HID), jnp.float32),
            pltpu.VMEM((CH, HID), jnp.float32),
            pltpu.SemaphoreType.DMA,
            pltpu.SemaphoreType.DMA,
            pltpu.SemaphoreType.DMA,
            pltpu.VMEM((2 * L,), jnp.float32),
            pltpu.VMEM((2 * L,), jnp.int32),
            pltpu.VMEM((2 * L,), jnp.float32),
            pltpu.VMEM((2 * L,), jnp.int32),
            pltpu.VMEM((2 * L,), jnp.int32),
            pltpu.VMEM((2 * L, HID), jnp.float32),
            pltpu.VMEM_SHARED((NS * L,), jnp.float32),
            pltpu.VMEM_SHARED((NS * 2 * L,), jnp.float32),
            pltpu.VMEM_SHARED((NS * 2 * L,), jnp.int32),
        ],
    )
    keys2, rows64 = phase1(query, memory_bank, centroids, cluster_ids)

    finish = pl.pallas_call(
        _finish_kernel,
        out_shape=jax.ShapeDtypeStruct((K, HID), jnp.float32),
    )
    return finish(keys2.reshape(2 * K, 1), keys2.reshape(1, 2 * K), rows64)


# speedup vs baseline: 1.0971x; 1.0346x over previous
import dataclasses

import jax
import jax.numpy as jnp
from jax import lax
from jax.experimental import pallas as pl
from jax.experimental.pallas import tpu as pltpu
from jax.experimental.pallas import tpu_sc as plsc

CAP = 50000
HID = 768
K = 32
NC = 2
NS = 16
L = 16
RPT = 1568
CH = 48
IDXBUF = RPT + CH
EUN = 16

INF = float("inf")
IMAX = 0x7FFFFFFF


def _iota():
    return lax.iota(jnp.int32, L)


def _sort16(k, v):
    return plsc.sort_key_val(k, v)


def _rev(x):
    return jnp.flip(x, 0)


def _cmpx(ak, av, bk, bv):
    m = ak <= bk
    lk = jnp.where(m, ak, bk)
    lv = jnp.where(m, av, bv)
    hk = jnp.where(m, bk, ak)
    hv = jnp.where(m, bv, av)
    return lk, lv, hk, hv


def _merge_batch(b0k, b0v, b1k, b1v, nk, nv):
    nk, nv = _sort16(nk, nv)
    l1k, l1v, _, _ = _cmpx(b1k, b1v, _rev(nk), _rev(nv))
    l1k, l1v = _sort16(l1k, l1v)
    l2k, l2v, h2k, h2v = _cmpx(b0k, b0v, _rev(l1k), _rev(l1v))
    l2k, l2v = _sort16(l2k, l2v)
    h2k, h2v = _sort16(h2k, h2v)
    return l2k, l2v, h2k, h2v


def _merge32(a0k, a0v, a1k, a1v, p0k, p0v, p1k, p1v):
    l0k, l0v, _, _ = _cmpx(a0k, a0v, _rev(p1k), _rev(p1v))
    l1k, l1v, _, _ = _cmpx(a1k, a1v, _rev(p0k), _rev(p0v))
    l0k, l0v = _sort16(l0k, l0v)
    l1k, l1v = _sort16(l1k, l1v)
    m0k, m0v, m1k, m1v = _cmpx(l0k, l0v, _rev(l1k), _rev(l1v))
    m0k, m0v = _sort16(m0k, m0v)
    m1k, m1v = _sort16(m1k, m1v)
    return m0k, m0v, m1k, m1v


def _row_d2_contig(buf, qv):
    acc = jnp.zeros((L,), jnp.float32)
    for c in range(HID // L):
        d = buf[pl.ds(c * L, L)] - qv[pl.ds(c * L, L)]
        acc = acc + d * d
    return jnp.sum(acc)


def _phase1_kernel(q_hbm, bank_hbm, cent_hbm, cid_hbm, outk_hbm, outr_hbm,
                   qv, cb0, cb1, cstage, clocal, kbuf, cidv, idxv,
                   rbufa, rbufb, sema, semb, semc,
                   mk, mi, pk, pi, iw, rbig, csh, mshk, mshi):
    c = lax.axis_index("c")
    s = lax.axis_index("s")
    wid = s * NC + c
    iota = _iota()

    lo = wid * RPT
    base = lax.min(lo, CAP - RPT)
    pltpu.async_copy(
        cid_hbm.at[pl.ds(pl.multiple_of(base, 8), RPT)], cidv, semc)
    pltpu.sync_copy(q_hbm, qv)

    pltpu.sync_copy(cent_hbm.at[2 * s], cb0)
    pltpu.sync_copy(cent_hbm.at[2 * s + 1], cb1)
    d2_0 = _row_d2_contig(cb0, qv)
    d2_1 = _row_d2_contig(cb1, qv)
    vrow = jnp.where(iota == 0, d2_0, jnp.where(iota == 1, d2_1, INF))
    cstage[...] = vrow
    pltpu.sync_copy(cstage, csh.at[pl.ds(pl.multiple_of(s * L, 8), L)])
    plsc.subcore_barrier()
    pltpu.sync_copy(csh, clocal)
    plsc.subcore_barrier()

    for s2 in range(NS):
        row = clocal[pl.ds(s2 * L, L)]
        key = lax.bitcast_convert_type(row, jnp.int32)
        key = jnp.bitwise_and(key, jnp.int32(~0x1F))
        key = jnp.bitwise_or(key, 2 * s2 + iota)
        key = jnp.where(iota < 2, key, IMAX)
        kbuf[pl.ds(s2 * L, L)] = key

    tsel = []
    for _ in range(3):
        km = kbuf[pl.ds(0, L)]
        for s2 in range(1, NS):
            km = jnp.minimum(km, kbuf[pl.ds(s2 * L, L)])
        kmin = jnp.min(km)
        tsel.append(jnp.bitwise_and(kmin, jnp.int32(0x1F)))
        for s2 in range(NS):
            row = kbuf[pl.ds(s2 * L, L)]
            kbuf[pl.ds(s2 * L, L)] = jnp.where(row == kmin, IMAX, row)
    t0, t1, t2 = tsel

    pltpu.make_async_copy(
        cid_hbm.at[pl.ds(pl.multiple_of(base, 8), RPT)], cidv, semc).wait()
    zeros16 = jnp.zeros((L,), jnp.int32)
    for j in range(IDXBUF // L):
        idxv[pl.ds(j * L, L)] = zeros16

    def comp_body(j, cnt):
        cvec = cidv[pl.ds(j * L, L)]
        gvec = base + j * L + iota
        m = (cvec == t0) | (cvec == t1) | (cvec == t2)
        m = m & (gvec >= lo)
        plsc.store_compressed(idxv.at[pl.ds(cnt, L)], gvec, mask=m)
        return cnt + jnp.sum(m.astype(jnp.int32))

    cnt = lax.fori_loop(0, RPT // L, comp_body, jnp.int32(0))

    infv = jnp.full((L,), INF, jnp.float32)
    zi = jnp.zeros((L,), jnp.int32)
    zf = jnp.zeros((L,), jnp.float32)
    nch = (cnt + (CH - 1)) // CH
    rows0 = iota
    rows1 = iota + L
    rows2 = iota + 2 * L

    def issue(g, rbuf, sem):
        off = pl.multiple_of(g * CH, 8)
        pltpu.async_copy(bank_hbm.at[idxv.at[pl.ds(off, CH)]], rbuf, sem)

    def wait(rbuf, sem):
        pltpu.make_async_copy(
            bank_hbm.at[idxv.at[pl.ds(0, CH)]], rbuf, sem).wait()

    def compute_merge(g, rbuf, carry):
        b0k, b0v, b1k, b1v = carry
        off = pl.multiple_of(g * CH, 8)

        for grp in range(CH // L):
            def cb(cblk, accs, grp=grp):
                accs = list(accs)
                qoff = pl.multiple_of(cblk * L, 8)
                qc = qv[pl.ds(qoff, L)]
                for r in range(L):
                    x = rbuf[grp * L + r, pl.ds(qoff, L)]
                    d = x - qc
                    accs[r] = accs[r] + d * d
                return tuple(accs)

            accs = lax.fori_loop(0, HID // L, cb, (zf,) * L)
            db = zf
            for r in range(L):
                db = jnp.where(iota == r, jnp.sum(accs[r]), db)
            ivec = idxv[pl.ds(pl.multiple_of(off + grp * L, 8), L)]
            pos = off + grp * L + iota
            dbatch = jnp.where(pos < cnt, db, INF)
            b0k, b0v, b1k, b1v = _merge_batch(b0k, b0v, b1k, b1v, dbatch, ivec)
        return (b0k, b0v, b1k, b1v)

    @pl.when(nch > 0)
    def _():
        issue(0, rbufa, sema)

    def pair_body(t, carry):
        g0 = 2 * t
        wait(rbufa, sema)

        @pl.when(g0 + 1 < nch)
        def _():
            issue(g0 + 1, rbufb, semb)

        carry = compute_merge(g0, rbufa, carry)

        def second(carry):
            wait(rbufb, semb)

            @pl.when(g0 + 2 < nch)
            def _():
                issue(g0 + 2, rbufa, sema)

            return compute_merge(g0 + 1, rbufb, carry)

        return lax.cond(g0 + 1 < nch, second, lambda op: op, carry)

    b0k, b0v, b1k, b1v = lax.fori_loop(
        0, (nch + 1) // 2, pair_body, (infv, zi, infv, zi))

    for r in (1, 2, 4, 8):
        mk[pl.ds(0, L)] = b0k
        mk[pl.ds(L, L)] = b1k
        mi[pl.ds(0, L)] = b0v
        mi[pl.ds(L, L)] = b1v
        pltpu.sync_copy(mk, mshk.at[pl.ds(pl.multiple_of(s * 2 * L, 8), 2 * L)])
        pltpu.sync_copy(mi, mshi.at[pl.ds(pl.multiple_of(s * 2 * L, 8), 2 * L)])
        plsc.subcore_barrier()

        def merged(operand, r=r):
            b0k, b0v, b1k, b1v = operand
            poff = pl.multiple_of((s + r) * 2 * L, 8)
            pltpu.sync_copy(mshk.at[pl.ds(poff, 2 * L)], pk)
            pltpu.sync_copy(mshi.at[pl.ds(poff, 2 * L)], pi)
            return _merge32(b0k, b0v, b1k, b1v,
                            pk[pl.ds(0, L)], pi[pl.ds(0, L)],
                            pk[pl.ds(L, L)], pi[pl.ds(L, L)])

        b0k, b0v, b1k, b1v = lax.cond(
            s % (2 * r) == 0, merged, lambda op: op, (b0k, b0v, b1k, b1v))
        plsc.subcore_barrier()

    @pl.when(s == 0)
    def _():
        mk[pl.ds(0, L)] = b0k
        mk[pl.ds(L, L)] = b1k
        iw[pl.ds(0, L)] = b0v
        iw[pl.ds(L, L)] = b1v
        off = pl.multiple_of(c * 2 * L, 8)
        pltpu.sync_copy(mk, outk_hbm.at[pl.ds(off, 2 * L)])
        pltpu.sync_copy(bank_hbm.at[iw], rbig)
        pltpu.sync_copy(rbig, outr_hbm.at[pl.ds(off, 2 * L)])


def _finish_kernel(kc_ref, kr_ref, rows_ref, out_ref):
    kc = kc_ref[...]
    kr = kr_ref[...]
    ic = lax.broadcasted_iota(jnp.int32, (2 * K, 2 * K), 0)
    ir = lax.broadcasted_iota(jnp.int32, (2 * K, 2 * K), 1)
    beats = (kr < kc) | ((kr == kc) & (ir < ic))
    rank = jnp.sum(beats.astype(jnp.int32), axis=1, keepdims=True)
    sel = lax.broadcasted_iota(jnp.int32, (2 * K, K), 1)
    onehot = (rank == sel).astype(jnp.float32)
    out_ref[...] = lax.dot_general(
        onehot, rows_ref[...], (((0,), (0,)), ((), ())),
        precision=lax.Precision.HIGHEST,
        preferred_element_type=jnp.float32)


def _sc_params():
    cp = pltpu.CompilerParams()
    if "needs_layout_passes" in pltpu.CompilerParams.__dataclass_fields__:
        cp = dataclasses.replace(cp, needs_layout_passes=False)
    return cp


def kernel(query, memory_bank, centroids, cluster_ids, k):
    del k
    mesh = plsc.VectorSubcoreMesh(core_axis_name="c", subcore_axis_name="s")

    phase1 = pl.kernel(
        _phase1_kernel,
        out_type=(
            jax.ShapeDtypeStruct((NC * 2 * L,), jnp.float32),
            jax.ShapeDtypeStruct((NC * 2 * L, HID), jnp.float32),
        ),
        mesh=mesh,
        compiler_params=_sc_params(),
        scratch_types=[
            pltpu.VMEM((HID,), jnp.float32),
            pltpu.VMEM((HID,), jnp.float32),
            pltpu.VMEM((HID,), jnp.float32),
            pltpu.VMEM((L,), jnp.float32),
            pltpu.VMEM((NS * L,), jnp.float32),
            pltpu.VMEM((NS * L,), jnp.int32),
            pltpu.VMEM((RPT,), jnp.int32),
            pltpu.VMEM((IDXBUF,), jnp.int32),
            pltpu.VMEM((CH, HID), jnp.float32),
            pltpu.VMEM((CH, HID), jnp.float32),
            pltpu.SemaphoreType.DMA,
            pltpu.SemaphoreType.DMA,
            pltpu.SemaphoreType.DMA,
            pltpu.VMEM((2 * L,), jnp.float32),
            pltpu.VMEM((2 * L,), jnp.int32),
            pltpu.VMEM((2 * L,), jnp.float32),
            pltpu.VMEM((2 * L,), jnp.int32),
            pltpu.VMEM((2 * L,), jnp.int32),
            pltpu.VMEM((2 * L, HID), jnp.float32),
            pltpu.VMEM_SHARED((NS * L,), jnp.float32),
            pltpu.VMEM_SHARED((NS * 2 * L,), jnp.float32),
            pltpu.VMEM_SHARED((NS * 2 * L,), jnp.int32),
        ],
    )
    keys2, rows64 = phase1(query, memory_bank, centroids, cluster_ids)

    finish = pl.pallas_call(
        _finish_kernel,
        out_shape=jax.ShapeDtypeStruct((K, HID), jnp.float32),
    )
    return finish(keys2.reshape(2 * K, 1), keys2.reshape(1, 2 * K), rows64)


# speedup vs baseline: 1.5917x; 1.4508x over previous
import dataclasses

import jax
import jax.numpy as jnp
from jax import lax
from jax.experimental import pallas as pl
from jax.experimental.pallas import tpu as pltpu
from jax.experimental.pallas import tpu_sc as plsc

CAP = 50000
HID = 768
K = 32
NC = 2
NS = 16
L = 16
RPT = 1568
CH = 48
IDXBUF = RPT + CH
EUN = 16

INF = float("inf")
IMAX = 0x7FFFFFFF


def _iota():
    return lax.iota(jnp.int32, L)


def _sort16(k, v):
    return plsc.sort_key_val(k, v)


def _rev(x):
    return jnp.flip(x, 0)


def _cmpx(ak, av, bk, bv):
    m = ak <= bk
    lk = jnp.where(m, ak, bk)
    lv = jnp.where(m, av, bv)
    hk = jnp.where(m, bk, ak)
    hv = jnp.where(m, bv, av)
    return lk, lv, hk, hv


def _merge_batch(b0k, b0v, b1k, b1v, nk, nv):
    nk, nv = _sort16(nk, nv)
    l1k, l1v, _, _ = _cmpx(b1k, b1v, _rev(nk), _rev(nv))
    l1k, l1v = _sort16(l1k, l1v)
    l2k, l2v, h2k, h2v = _cmpx(b0k, b0v, _rev(l1k), _rev(l1v))
    l2k, l2v = _sort16(l2k, l2v)
    h2k, h2v = _sort16(h2k, h2v)
    return l2k, l2v, h2k, h2v


def _merge32(a0k, a0v, a1k, a1v, p0k, p0v, p1k, p1v):
    l0k, l0v, _, _ = _cmpx(a0k, a0v, _rev(p1k), _rev(p1v))
    l1k, l1v, _, _ = _cmpx(a1k, a1v, _rev(p0k), _rev(p0v))
    l0k, l0v = _sort16(l0k, l0v)
    l1k, l1v = _sort16(l1k, l1v)
    m0k, m0v, m1k, m1v = _cmpx(l0k, l0v, _rev(l1k), _rev(l1v))
    m0k, m0v = _sort16(m0k, m0v)
    m1k, m1v = _sort16(m1k, m1v)
    return m0k, m0v, m1k, m1v


def _row_d2_contig(buf, qv):
    acc = jnp.zeros((L,), jnp.float32)
    for c in range(HID // L):
        d = buf[pl.ds(c * L, L)] - qv[pl.ds(c * L, L)]
        acc = acc + d * d
    return jnp.sum(acc)


def _phase1_kernel(q_hbm, bank_hbm, cent_hbm, cid_hbm, outk_hbm, outr_hbm,
                   qv, cb0, cb1, cstage, clocal, kbuf, cidv, idxv,
                   rbufa, rbufb, sema, semb, semc,
                   mk, mi, pk, pi, iw, rbig, csh, mshk, mshi):
    c = lax.axis_index("c")
    s = lax.axis_index("s")
    wid = s * NC + c
    iota = _iota()

    lo = wid * RPT
    base = lax.min(lo, CAP - RPT)
    pltpu.async_copy(
        cid_hbm.at[pl.ds(pl.multiple_of(base, 8), RPT)], cidv, semc)
    pltpu.sync_copy(q_hbm, qv)

    pltpu.sync_copy(cent_hbm.at[2 * s], cb0)
    pltpu.sync_copy(cent_hbm.at[2 * s + 1], cb1)
    d2_0 = _row_d2_contig(cb0, qv)
    d2_1 = _row_d2_contig(cb1, qv)
    vrow = jnp.where(iota == 0, d2_0, jnp.where(iota == 1, d2_1, INF))
    cstage[...] = vrow
    pltpu.sync_copy(cstage, csh.at[pl.ds(pl.multiple_of(s * L, 8), L)])
    plsc.subcore_barrier()
    pltpu.sync_copy(csh, clocal)
    plsc.subcore_barrier()

    for s2 in range(NS):
        row = clocal[pl.ds(s2 * L, L)]
        key = lax.bitcast_convert_type(row, jnp.int32)
        key = jnp.bitwise_and(key, jnp.int32(~0x1F))
        key = jnp.bitwise_or(key, 2 * s2 + iota)
        key = jnp.where(iota < 2, key, IMAX)
        kbuf[pl.ds(s2 * L, L)] = key

    tsel = []
    for _ in range(3):
        km = kbuf[pl.ds(0, L)]
        for s2 in range(1, NS):
            km = jnp.minimum(km, kbuf[pl.ds(s2 * L, L)])
        kmin = jnp.min(km)
        tsel.append(jnp.bitwise_and(kmin, jnp.int32(0x1F)))
        for s2 in range(NS):
            row = kbuf[pl.ds(s2 * L, L)]
            kbuf[pl.ds(s2 * L, L)] = jnp.where(row == kmin, IMAX, row)
    t0, t1, t2 = tsel

    pltpu.make_async_copy(
        cid_hbm.at[pl.ds(pl.multiple_of(base, 8), RPT)], cidv, semc).wait()
    zeros16 = jnp.zeros((L,), jnp.int32)
    for j in range(IDXBUF // L):
        idxv[pl.ds(j * L, L)] = lax.min(base + j * L + iota, CAP - 1)

    def comp_body(j, cnt):
        cvec = cidv[pl.ds(j * L, L)]
        gvec = base + j * L + iota
        m = (cvec == t0) | (cvec == t1) | (cvec == t2)
        m = m & (gvec >= lo)
        plsc.store_compressed(idxv.at[pl.ds(cnt, L)], gvec, mask=m)
        return cnt + jnp.sum(m.astype(jnp.int32))

    cnt = lax.fori_loop(0, RPT // L, comp_body, jnp.int32(0))

    infv = jnp.full((L,), INF, jnp.float32)
    zi = jnp.zeros((L,), jnp.int32)
    zf = jnp.zeros((L,), jnp.float32)
    nch = (cnt + (CH - 1)) // CH
    rows0 = iota
    rows1 = iota + L
    rows2 = iota + 2 * L

    def issue(g, rbuf, sem):
        off = pl.multiple_of(g * CH, 8)
        pltpu.async_copy(bank_hbm.at[idxv.at[pl.ds(off, CH)]], rbuf, sem)

    def wait(rbuf, sem):
        pltpu.make_async_copy(
            bank_hbm.at[idxv.at[pl.ds(0, CH)]], rbuf, sem).wait()

    def compute_merge(g, rbuf, carry):
        b0k, b0v, b1k, b1v = carry
        off = pl.multiple_of(g * CH, 8)

        for grp in range(CH // L):
            def cb(cblk, accs, grp=grp):
                accs = list(accs)
                for u in range(4):
                    qoff = pl.multiple_of(cblk * 4 * L + u * L, 8)
                    qc = qv[pl.ds(qoff, L)]
                    for r in range(L):
                        x = rbuf[grp * L + r, pl.ds(qoff, L)]
                        d = x - qc
                        accs[r] = accs[r] + d * d
                return tuple(accs)

            accs = lax.fori_loop(0, HID // (4 * L), cb, (zf,) * L)
            db = zf
            for r in range(L):
                db = jnp.where(iota == r, jnp.sum(accs[r]), db)
            ivec = idxv[pl.ds(pl.multiple_of(off + grp * L, 8), L)]
            pos = off + grp * L + iota
            dbatch = jnp.where(pos < cnt, db, INF)
            b0k, b0v, b1k, b1v = _merge_batch(b0k, b0v, b1k, b1v, dbatch, ivec)
        return (b0k, b0v, b1k, b1v)

    @pl.when(nch > 0)
    def _():
        issue(0, rbufa, sema)

    def pair_body(t, carry):
        g0 = 2 * t
        wait(rbufa, sema)

        @pl.when(g0 + 1 < nch)
        def _():
            issue(g0 + 1, rbufb, semb)

        carry = compute_merge(g0, rbufa, carry)

        def second(carry):
            wait(rbufb, semb)

            @pl.when(g0 + 2 < nch)
            def _():
                issue(g0 + 2, rbufa, sema)

            return compute_merge(g0 + 1, rbufb, carry)

        return lax.cond(g0 + 1 < nch, second, lambda op: op, carry)

    b0k, b0v, b1k, b1v = lax.fori_loop(
        0, (nch + 1) // 2, pair_body, (infv, zi, infv, zi))

    for r in (1, 2, 4, 8):
        mk[pl.ds(0, L)] = b0k
        mk[pl.ds(L, L)] = b1k
        mi[pl.ds(0, L)] = b0v
        mi[pl.ds(L, L)] = b1v
        pltpu.sync_copy(mk, mshk.at[pl.ds(pl.multiple_of(s * 2 * L, 8), 2 * L)])
        pltpu.sync_copy(mi, mshi.at[pl.ds(pl.multiple_of(s * 2 * L, 8), 2 * L)])
        plsc.subcore_barrier()

        def merged(operand, r=r):
            b0k, b0v, b1k, b1v = operand
            poff = pl.multiple_of((s + r) * 2 * L, 8)
            pltpu.sync_copy(mshk.at[pl.ds(poff, 2 * L)], pk)
            pltpu.sync_copy(mshi.at[pl.ds(poff, 2 * L)], pi)
            return _merge32(b0k, b0v, b1k, b1v,
                            pk[pl.ds(0, L)], pi[pl.ds(0, L)],
                            pk[pl.ds(L, L)], pi[pl.ds(L, L)])

        b0k, b0v, b1k, b1v = lax.cond(
            s % (2 * r) == 0, merged, lambda op: op, (b0k, b0v, b1k, b1v))
        plsc.subcore_barrier()

    @pl.when(s == 0)
    def _():
        mk[pl.ds(0, L)] = b0k
        mk[pl.ds(L, L)] = b1k
        iw[pl.ds(0, L)] = b0v
        iw[pl.ds(L, L)] = b1v
        off = pl.multiple_of(c * 2 * L, 8)
        pltpu.sync_copy(mk, outk_hbm.at[pl.ds(off, 2 * L)])
        pltpu.sync_copy(bank_hbm.at[iw], rbig)
        pltpu.sync_copy(rbig, outr_hbm.at[pl.ds(off, 2 * L)])


def _finish_kernel(kc_ref, kr_ref, rows_ref, out_ref):
    kc = kc_ref[...]
    kr = kr_ref[...]
    ic = lax.broadcasted_iota(jnp.int32, (2 * K, 2 * K), 0)
    ir = lax.broadcasted_iota(jnp.int32, (2 * K, 2 * K), 1)
    beats = (kr < kc) | ((kr == kc) & (ir < ic))
    rank = jnp.sum(beats.astype(jnp.int32), axis=1, keepdims=True)
    sel = lax.broadcasted_iota(jnp.int32, (2 * K, K), 1)
    onehot = (rank == sel).astype(jnp.float32)
    out_ref[...] = lax.dot_general(
        onehot, rows_ref[...], (((0,), (0,)), ((), ())),
        precision=lax.Precision.HIGHEST,
        preferred_element_type=jnp.float32)


def _sc_params():
    cp = pltpu.CompilerParams()
    if "needs_layout_passes" in pltpu.CompilerParams.__dataclass_fields__:
        cp = dataclasses.replace(cp, needs_layout_passes=False)
    return cp


def kernel(query, memory_bank, centroids, cluster_ids, k):
    del k
    mesh = plsc.VectorSubcoreMesh(core_axis_name="c", subcore_axis_name="s")

    phase1 = pl.kernel(
        _phase1_kernel,
        out_type=(
            jax.ShapeDtypeStruct((NC * 2 * L,), jnp.float32),
            jax.ShapeDtypeStruct((NC * 2 * L, HID), jnp.float32),
        ),
        mesh=mesh,
        compiler_params=_sc_params(),
        scratch_types=[
            pltpu.VMEM((HID,), jnp.float32),
            pltpu.VMEM((HID,), jnp.float32),
            pltpu.VMEM((HID,), jnp.float32),
            pltpu.VMEM((L,), jnp.float32),
            pltpu.VMEM((NS * L,), jnp.float32),
            pltpu.VMEM((NS * L,), jnp.int32),
            pltpu.VMEM((RPT,), jnp.int32),
            pltpu.VMEM((IDXBUF,), jnp.int32),
            pltpu.VMEM((CH, HID), jnp.float32),
            pltpu.VMEM((CH, HID), jnp.float32),
            pltpu.SemaphoreType.DMA,
            pltpu.SemaphoreType.DMA,
            pltpu.SemaphoreType.DMA,
            pltpu.VMEM((2 * L,), jnp.float32),
            pltpu.VMEM((2 * L,), jnp.int32),
            pltpu.VMEM((2 * L,), jnp.float32),
            pltpu.VMEM((2 * L,), jnp.int32),
            pltpu.VMEM((2 * L,), jnp.int32),
            pltpu.VMEM((2 * L, HID), jnp.float32),
            pltpu.VMEM_SHARED((NS * L,), jnp.float32),
            pltpu.VMEM_SHARED((NS * 2 * L,), jnp.float32),
            pltpu.VMEM_SHARED((NS * 2 * L,), jnp.int32),
        ],
    )
    keys2, rows64 = phase1(query, memory_bank, centroids, cluster_ids)

    finish = pl.pallas_call(
        _finish_kernel,
        out_shape=jax.ShapeDtypeStruct((K, HID), jnp.float32),
    )
    return finish(keys2.reshape(2 * K, 1), keys2.reshape(1, 2 * K), rows64)
